# Initial kernel scaffold; baseline (speedup 1.0000x reference)
import functools

import jax
import jax.numpy as jnp
from jax.experimental import pallas as pl

_NEG = jnp.float32(-1e30)


def _attn_topk_block(x_ref, wbd_ref, b2_ref, abdt_ref, o_ref):
    P, NB2, D2 = x_ref.shape
    x = x_ref[...]
    xf = x.reshape(P * NB2, D2)
    h = jnp.tanh(
        jnp.dot(xf, wbd_ref[...], preferred_element_type=jnp.float32)
        + b2_ref[...]
    )
    h3 = h.reshape(P, NB2, D2)
    lps = [
        jax.lax.dot_general(
            abdt_ref[...], h3[p], (((1,), (1,)), ((), ())),
            preferred_element_type=jnp.float32,
        )
        for p in range(P)
    ]
    l = jnp.stack(lps, axis=0)

    pidx = jax.lax.broadcasted_iota(jnp.int32, l.shape, 0)
    m1 = jnp.max(l, axis=0, keepdims=True)
    idx1 = jnp.min(jnp.where(l == m1, pidx, P), axis=0, keepdims=True)
    sel1 = pidx == idx1
    l2 = jnp.where(sel1, _NEG, l)
    m2 = jnp.max(l2, axis=0, keepdims=True)
    idx2 = jnp.min(jnp.where(l2 == m2, pidx, P), axis=0, keepdims=True)
    sel2 = pidx == idx2

    e = jnp.exp(l - m1)
    denom = jnp.sum(e, axis=0, keepdims=True)
    w = e / denom
    wsel = jnp.where(sel1 | sel2, w, jnp.float32(0.0))

    ridx = jax.lax.broadcasted_iota(jnp.int32, (2, D2), 0)
    lidx = jax.lax.broadcasted_iota(jnp.int32, (2, D2), 1)
    expand = jnp.where((lidx // (D2 // 2)) == ridx, jnp.float32(1.0),
                       jnp.float32(0.0))
    acc = jnp.zeros((NB2, D2), jnp.float32)
    for p in range(P):
        wx = jax.lax.dot_general(
            wsel[p], expand, (((0,), (0,)), ((), ())),
            preferred_element_type=jnp.float32,
        )
        acc = acc + wx * x[p]
    o_ref[...] = acc


@functools.partial(jax.jit, static_argnames=("interpret",))
def kernel(semantic_embeddings, attnVec, fc_w, fc_b, interpret=False):
    P, N, D = semantic_embeddings.shape
    x2 = semantic_embeddings.reshape(P, N // 2, 2 * D)
    a = attnVec.reshape(D)
    wT = fc_w.T
    zz = jnp.zeros((D, D), fc_w.dtype)
    wbd = jnp.concatenate(
        [jnp.concatenate([wT, zz], axis=1), jnp.concatenate([zz, wT], axis=1)],
        axis=0,
    )
    b2 = jnp.concatenate([fc_b, fc_b]).reshape(1, 2 * D)
    za = jnp.zeros((D,), a.dtype)
    abdt = jnp.stack(
        [jnp.concatenate([a, za]), jnp.concatenate([za, a])], axis=0
    )

    NB2 = 1000
    grid = (N // 2 // NB2,)
    out2 = pl.pallas_call(
        _attn_topk_block,
        grid=grid,
        in_specs=[
            pl.BlockSpec((P, NB2, 2 * D), lambda i: (0, i, 0)),
            pl.BlockSpec((2 * D, 2 * D), lambda i: (0, 0)),
            pl.BlockSpec((1, 2 * D), lambda i: (0, 0)),
            pl.BlockSpec((2, 2 * D), lambda i: (0, 0)),
        ],
        out_specs=pl.BlockSpec((NB2, 2 * D), lambda i: (i, 0)),
        out_shape=jax.ShapeDtypeStruct((N // 2, 2 * D), jnp.float32),
        interpret=interpret,
    )(x2, wbd, b2, abdt)
    return out2.reshape(N, D)


# speedup vs baseline: 1.7983x; 1.7983x over previous
import functools

import jax
import jax.numpy as jnp
from jax.experimental import pallas as pl

_F = 4
_NB = 1000


def _attn_topk_block(x_ref, wbd_ref, b2_ref, abdt_ref, o_ref):
    P, NB, DF = x_ref.shape
    F = abdt_ref.shape[0]
    D = DF // F
    x = x_ref[...]
    xf = x.reshape(P * NB, DF)
    KW = wbd_ref.shape[0]
    z = jnp.concatenate(
        [
            jnp.dot(xf[:, j * KW:(j + 1) * KW], wbd_ref[...],
                    preferred_element_type=jnp.float32)
            for j in range(DF // KW)
        ],
        axis=1,
    )
    h = jnp.tanh(z + b2_ref[...])
    h3 = h.reshape(P, NB, DF)
    abdt = abdt_ref[...]
    dims = (((1,), (1,)), ((), ()))
    lps = [
        jax.lax.dot_general(abdt, h3[p], dims,
                            preferred_element_type=jnp.float32)
        for p in range(P)
    ]
    l = jnp.stack(lps, axis=0)

    pidx = jax.lax.broadcasted_iota(jnp.int32, l.shape, 0)
    m1 = jnp.max(l, axis=0, keepdims=True)
    idx1 = jnp.min(jnp.where(l == m1, pidx, P), axis=0, keepdims=True)
    sel1 = pidx == idx1
    l2 = jnp.where(sel1, -1e30, l)
    m2 = jnp.max(l2, axis=0, keepdims=True)
    idx2 = jnp.min(jnp.where(l2 == m2, pidx, P), axis=0, keepdims=True)
    sel2 = pidx == idx2

    e = jnp.exp(l - m1)
    denom = jnp.sum(e, axis=0, keepdims=True)
    w = e / denom
    wsel = jnp.where(sel1 | sel2, w, jnp.float32(0.0))

    ridx = jax.lax.broadcasted_iota(jnp.int32, (F, DF), 0)
    lidx = jax.lax.broadcasted_iota(jnp.int32, (F, DF), 1)
    expand = jnp.where((lidx // D) == ridx, jnp.float32(1.0),
                       jnp.float32(0.0))
    acc = jnp.zeros((NB, DF), jnp.float32)
    for p in range(P):
        wsel_t = jnp.transpose(wsel[p])
        wx = jax.lax.dot_general(
            wsel_t, expand, (((1,), (0,)), ((), ())),
            preferred_element_type=jnp.float32,
        )
        acc = acc + wx * x[p]
    o_ref[...] = acc


@functools.partial(jax.jit, static_argnames=("interpret",))
def kernel(semantic_embeddings, attnVec, fc_w, fc_b, interpret=False):
    P, N, D = semantic_embeddings.shape
    F, NB = _F, _NB
    x2 = semantic_embeddings.reshape(P, N // F, F * D)
    a = attnVec.reshape(1, D)
    eyeF = jnp.eye(F, dtype=jnp.float32)
    wbd = jnp.kron(jnp.eye(2, dtype=jnp.float32), fc_w.T)
    b2 = jnp.tile(fc_b, F).reshape(1, F * D)
    abdt = jnp.kron(eyeF, a)

    grid = (N // F // NB,)
    out2 = pl.pallas_call(
        _attn_topk_block,
        grid=grid,
        in_specs=[
            pl.BlockSpec((P, NB, F * D), lambda i: (0, i, 0)),
            pl.BlockSpec((2 * D, 2 * D), lambda i: (0, 0)),
            pl.BlockSpec((1, F * D), lambda i: (0, 0)),
            pl.BlockSpec((F, F * D), lambda i: (0, 0)),
        ],
        out_specs=pl.BlockSpec((NB, F * D), lambda i: (i, 0)),
        out_shape=jax.ShapeDtypeStruct((N // F, F * D), jnp.float32),
        interpret=interpret,
    )(x2, wbd, b2, abdt)
    return out2.reshape(N, D)


# speedup vs baseline: 2.2378x; 1.2444x over previous
import functools

import jax
import jax.numpy as jnp
from jax.experimental import pallas as pl

_NB = 4000


def _attn_topk_block(x_ref, wt_ref, b_ref, a2_ref, o_ref):
    P, NB, D = x_ref.shape
    x = x_ref[...]
    xf = x.reshape(P * NB, D)
    z = jnp.dot(xf, wt_ref[...], preferred_element_type=jnp.float32)
    h = jnp.tanh(z + b_ref[...])
    h3 = h.reshape(P, NB, D)
    a2 = a2_ref[...]
    dims = (((1,), (1,)), ((), ()))
    l = jnp.concatenate(
        [
            jax.lax.dot_general(a2, h3[p], dims,
                                preferred_element_type=jnp.float32)
            for p in range(P)
        ],
        axis=0,
    )

    pidx = jax.lax.broadcasted_iota(jnp.int32, l.shape, 0)
    m1 = jnp.max(l, axis=0, keepdims=True)
    idx1 = jnp.min(jnp.where(l == m1, pidx, P), axis=0, keepdims=True)
    sel1 = pidx == idx1
    l2 = jnp.where(sel1, -1e30, l)
    m2 = jnp.max(l2, axis=0, keepdims=True)
    idx2 = jnp.min(jnp.where(l2 == m2, pidx, P), axis=0, keepdims=True)
    sel2 = pidx == idx2

    e = jnp.exp(l - m1)
    denom = jnp.sum(e, axis=0, keepdims=True)
    w = e / denom
    wsel = jnp.where(sel1 | sel2, w, jnp.float32(0.0))

    ones_row = jnp.full((1, D), 1.0, jnp.float32)
    acc = jnp.zeros((NB, D), jnp.float32)
    for p in range(P):
        wsel_t = jnp.transpose(wsel[p:p + 1])
        wx = jax.lax.dot_general(
            wsel_t, ones_row, (((1,), (0,)), ((), ())),
            preferred_element_type=jnp.float32,
        )
        acc = acc + wx * x[p]
    o_ref[...] = acc


@functools.partial(jax.jit, static_argnames=("interpret",))
def kernel(semantic_embeddings, attnVec, fc_w, fc_b, interpret=False):
    P, N, D = semantic_embeddings.shape
    NB = _NB
    a2 = attnVec.reshape(1, D)
    b2 = fc_b.reshape(1, D)

    grid = (N // NB,)
    out = pl.pallas_call(
        _attn_topk_block,
        grid=grid,
        in_specs=[
            pl.BlockSpec((P, NB, D), lambda i: (0, i, 0)),
            pl.BlockSpec((D, D), lambda i: (0, 0)),
            pl.BlockSpec((1, D), lambda i: (0, 0)),
            pl.BlockSpec((1, D), lambda i: (0, 0)),
        ],
        out_specs=pl.BlockSpec((NB, D), lambda i: (i, 0)),
        out_shape=jax.ShapeDtypeStruct((N, D), jnp.float32),
        interpret=interpret,
    )(semantic_embeddings, fc_w.T, b2, a2)
    return out
